# baseline (speedup 1.0000x reference)
import jax
import jax.numpy as jnp
from jax import lax
from jax.experimental import pallas as pl
from jax.experimental.pallas import tpu as pltpu
from jax.experimental.pallas import tpu_sc as plsc

B = 16384
C = 7
NW = 32
SPW = B // NW
CHUNKS = SPW // 16
NBINS = 64
EPS = 0.001


def _codes_body(x_ref, t_ref, o_ref):
    xi = x_ref[...]
    ti = t_ref[...]
    i7 = lax.broadcasted_iota(jnp.int32, (C, B), 0)

    def amax(v):
        m = jnp.max(v, axis=0, keepdims=True)
        return jnp.min(jnp.where(v == m, i7, C), axis=0)

    p = amax(xi)
    t = amax(ti)
    o_ref[...] = p * 8 + t


_codes = pl.pallas_call(
    _codes_body,
    out_shape=jax.ShapeDtypeStruct((B,), jnp.int32),
    in_specs=[pl.BlockSpec((C, B), lambda: (0, 0)),
              pl.BlockSpec((C, B), lambda: (0, 0))],
    out_specs=pl.BlockSpec((B,), lambda: (0,)),
)


def _sc_body(codes_hbm, out_hbm, codes_v, hist_v, red_v):
    sid = lax.axis_index("s") * 2 + lax.axis_index("c")
    pltpu.sync_copy(codes_hbm.at[pl.ds(sid * SPW, SPW)], codes_v)

    zeros = jnp.zeros((16,), jnp.float32)

    def zinit(k, carry):
        hist_v[pl.ds(k * 16, 16)] = zeros
        return carry

    lax.fori_loop(0, NBINS, zinit, 0)

    lane = lax.broadcasted_iota(jnp.int32, (16,), 0)
    lane_hist = lane * NBINS
    ones = jnp.ones((16,), jnp.float32)

    def chunk(j, carry):
        flat = codes_v[pl.ds(j * 16, 16)]
        plsc.addupdate_scatter(hist_v, [lane_hist + flat], ones)
        return carry

    lax.fori_loop(0, CHUNKS, chunk, 0)

    for c in range(4):
        def lred(l, acc):
            return acc + hist_v[pl.ds(l * 64 + c * 16, 16)]
        acc = lax.fori_loop(1, 16, lred, hist_v[pl.ds(c * 16, 16)])
        red_v[pl.ds(c * 16, 16)] = acc
    pltpu.sync_copy(red_v, out_hbm.at[sid])


_sc_cache = []


def _sc_call(codes):
    if not _sc_cache:
        _sc_cache.append(pl.kernel(
            _sc_body,
            out_type=jax.ShapeDtypeStruct((NW, NBINS), jnp.float32),
            mesh=plsc.VectorSubcoreMesh(
                core_axis_name="c", subcore_axis_name="s"),
            compiler_params=pltpu.CompilerParams(needs_layout_passes=False),
            scratch_types=[
                pltpu.VMEM((SPW,), jnp.int32),
                pltpu.VMEM((16 * NBINS,), jnp.float32),
                pltpu.VMEM((NBINS,), jnp.float32),
            ],
        ))
    return _sc_cache[0](codes)


def _bf(v):
    return v.astype(jnp.bfloat16).astype(jnp.float32)


def _tc_solve_body(h_ref, o_ref):
    x = h_ref[...]
    hh = jnp.sum(x, axis=0, keepdims=True)
    lanes = lax.broadcasted_iota(jnp.int32, (1, NBINS), 1)

    def ext(k):
        return jnp.sum(jnp.where(lanes == k, hh, 0.0), axis=(0, 1), keepdims=True)

    one = jnp.ones((1, 1), jnp.float32)
    epsv = jnp.full((1, 1), EPS, jnp.float32)

    cnt = [[ext(p * 8 + t) for t in range(7)] for p in range(1, 7)]
    cnt_bf = [[_bf(c) for c in row] for row in cnt]

    def row7(row):
        s = row[0]
        for t in range(1, 7):
            s = s + row[t]
        return s

    O = [row7(cnt[b]) for b in range(6)]
    cx = [row7(cnt_bf[b]) for b in range(6)]
    r = []
    for b in range(6):
        cz = cx[b] + epsv
        r.append(one / (cz * cz))

    A_bf = [[cnt_bf[y][xx + 1] for xx in range(6)] for y in range(6)]
    ej = [[_bf((A_bf[y][xx] * O[y]) * r[y]) for y in range(6)] for xx in range(6)]

    a = [[None] * 6 for _ in range(6)]
    for j in range(6):
        for k in range(6):
            acc = ej[k][0] * A_bf[0][j]
            for y in range(1, 6):
                acc = acc + ej[k][y] * A_bf[y][j]
            a[j][k] = acc

    iconst = [jnp.full((1, 1), v, jnp.int32) for v in range(6)]
    perm = [iconst[rr] for rr in range(6)]
    for k in range(6):
        mag = [jnp.abs(a[rr][k]) for rr in range(6)]
        best = mag[k]
        bi = iconst[k]
        for rr in range(k + 1, 6):
            gt = mag[rr] > best
            best = jnp.where(gt, mag[rr], best)
            bi = jnp.where(gt, iconst[rr], bi)
        rowk_old = [a[k][j] for j in range(6)]
        permk_old = perm[k]
        newk = []
        for j in range(6):
            v = a[k][j]
            for rr in range(k + 1, 6):
                v = jnp.where(bi == iconst[rr], a[rr][j], v)
            newk.append(v)
        newpk = perm[k]
        for rr in range(k + 1, 6):
            newpk = jnp.where(bi == iconst[rr], perm[rr], newpk)
        for rr in range(k + 1, 6):
            hit = bi == iconst[rr]
            for j in range(6):
                a[rr][j] = jnp.where(hit, rowk_old[j], a[rr][j])
            perm[rr] = jnp.where(hit, permk_old, perm[rr])
        a[k] = newk
        perm[k] = newpk
        xp = a[k][k]
        nz = xp != jnp.zeros((1, 1), jnp.float32)
        for rr in range(k + 1, 6):
            a[rr][k] = jnp.where(nz, a[rr][k] / xp, a[rr][k])
        for rr in range(k + 1, 6):
            for j in range(k + 1, 6):
                a[rr][j] = a[rr][j] - a[rr][k] * a[k][j]

    zero = jnp.zeros((1, 1), jnp.float32)
    Bm = [[jnp.where(perm[j] == iconst[c], one, zero) for c in range(6)]
          for j in range(6)]
    y = [[None] * 6 for _ in range(6)]
    for c in range(6):
        for rr in range(6):
            v = Bm[rr][c]
            for j in range(rr):
                v = v - a[rr][j] * y[j][c]
            y[rr][c] = v
    xs = [[None] * 6 for _ in range(6)]
    for c in range(6):
        for rr in range(5, -1, -1):
            v = y[rr][c]
            for j in range(rr + 1, 6):
                v = v - a[rr][j] * xs[j][c]
            xs[rr][c] = v / a[rr][rr]

    trace = xs[0][0]
    for c in range(1, 6):
        trace = trace + xs[c][c]
    o_ref[...] = lax.pow(trace, jnp.full((1, 1), 0.5, jnp.float32))


_tc_solve = pl.pallas_call(
    _tc_solve_body,
    out_shape=jax.ShapeDtypeStruct((1, 1), jnp.float32),
    in_specs=[pl.BlockSpec((NW, NBINS), lambda: (0, 0))],
    out_specs=pl.BlockSpec((1, 1), lambda: (0, 0)),
)


def kernel(input, target):
    codes = _codes(jnp.swapaxes(input, 0, 1), jnp.swapaxes(target, 0, 1))
    hist = _sc_call(codes)
    out = _tc_solve(hist)
    return lax.stop_gradient(out[0, 0])


# speedup vs baseline: 1.0567x; 1.0567x over previous
import jax
import jax.numpy as jnp
from jax import lax
from jax.experimental import pallas as pl
from jax.experimental.pallas import tpu as pltpu
from jax.experimental.pallas import tpu_sc as plsc

B = 16384
C = 7
NW = 16
SPW = B // NW
CHUNKS = SPW // 16
NBINS = 64
EPS = 0.001


def _codes_body(x_ref, t_ref, o_ref):
    xi = x_ref[...]
    ti = t_ref[...]
    i7 = lax.broadcasted_iota(jnp.int32, (C, B), 0)

    def amax(v):
        m = jnp.max(v, axis=0, keepdims=True)
        return jnp.min(jnp.where(v == m, i7, C), axis=0)

    p = amax(xi)
    t = amax(ti)
    o_ref[...] = p * 8 + t


_codes = pl.pallas_call(
    _codes_body,
    out_shape=jax.ShapeDtypeStruct((B,), jnp.int32),
    in_specs=[pl.BlockSpec((C, B), lambda: (0, 0)),
              pl.BlockSpec((C, B), lambda: (0, 0))],
    out_specs=pl.BlockSpec((B,), lambda: (0,)),
)


def _sc_body(codes_hbm, out_hbm, codes_v, hist_v, red_v):
    sid = lax.axis_index("s")
    pltpu.sync_copy(codes_hbm.at[pl.ds(sid * SPW, SPW)], codes_v)

    zeros = jnp.zeros((16,), jnp.float32)

    def zinit(k, carry):
        hist_v[pl.ds(k * 16, 16)] = zeros
        return carry

    lax.fori_loop(0, NBINS, zinit, 0)

    lane = lax.broadcasted_iota(jnp.int32, (16,), 0)
    lane_hist = lane * NBINS
    ones = jnp.ones((16,), jnp.float32)

    def chunk(j, carry):
        flat = codes_v[pl.ds(j * 16, 16)]
        plsc.addupdate_scatter(hist_v, [lane_hist + flat], ones)
        return carry

    lax.fori_loop(0, CHUNKS, chunk, 0)

    for c in range(4):
        def lred(l, acc):
            return acc + hist_v[pl.ds(l * 64 + c * 16, 16)]
        acc = lax.fori_loop(1, 16, lred, hist_v[pl.ds(c * 16, 16)])
        red_v[pl.ds(c * 16, 16)] = acc
    pltpu.sync_copy(red_v, out_hbm.at[sid])


_sc_cache = []


def _sc_call(codes):
    if not _sc_cache:
        _sc_cache.append(pl.kernel(
            _sc_body,
            out_type=jax.ShapeDtypeStruct((NW, NBINS), jnp.float32),
            mesh=plsc.VectorSubcoreMesh(
                core_axis_name="c", subcore_axis_name="s", num_cores=1),
            compiler_params=pltpu.CompilerParams(needs_layout_passes=False),
            scratch_types=[
                pltpu.VMEM((SPW,), jnp.int32),
                pltpu.VMEM((16 * NBINS,), jnp.float32),
                pltpu.VMEM((NBINS,), jnp.float32),
            ],
        ))
    return _sc_cache[0](codes)


def _bf(v):
    return v.astype(jnp.bfloat16).astype(jnp.float32)


def _tc_solve_body(h_ref, o_ref):
    x = h_ref[...]
    hh = jnp.sum(x, axis=0, keepdims=True)
    lanes = lax.broadcasted_iota(jnp.int32, (1, NBINS), 1)

    def ext(k):
        return jnp.sum(jnp.where(lanes == k, hh, 0.0), axis=(0, 1), keepdims=True)

    one = jnp.ones((1, 1), jnp.float32)
    epsv = jnp.full((1, 1), EPS, jnp.float32)

    cnt = [[ext(p * 8 + t) for t in range(7)] for p in range(1, 7)]
    cnt_bf = [[_bf(c) for c in row] for row in cnt]

    def row7(row):
        s = row[0]
        for t in range(1, 7):
            s = s + row[t]
        return s

    O = [row7(cnt[b]) for b in range(6)]
    cx = [row7(cnt_bf[b]) for b in range(6)]
    r = []
    for b in range(6):
        cz = cx[b] + epsv
        r.append(one / (cz * cz))

    A_bf = [[cnt_bf[y][xx + 1] for xx in range(6)] for y in range(6)]
    ej = [[_bf((A_bf[y][xx] * O[y]) * r[y]) for y in range(6)] for xx in range(6)]

    a = [[None] * 6 for _ in range(6)]
    for j in range(6):
        for k in range(6):
            acc = ej[k][0] * A_bf[0][j]
            for y in range(1, 6):
                acc = acc + ej[k][y] * A_bf[y][j]
            a[j][k] = acc

    iconst = [jnp.full((1, 1), v, jnp.int32) for v in range(6)]
    perm = [iconst[rr] for rr in range(6)]
    for k in range(6):
        mag = [jnp.abs(a[rr][k]) for rr in range(6)]
        best = mag[k]
        bi = iconst[k]
        for rr in range(k + 1, 6):
            gt = mag[rr] > best
            best = jnp.where(gt, mag[rr], best)
            bi = jnp.where(gt, iconst[rr], bi)
        rowk_old = [a[k][j] for j in range(6)]
        permk_old = perm[k]
        newk = []
        for j in range(6):
            v = a[k][j]
            for rr in range(k + 1, 6):
                v = jnp.where(bi == iconst[rr], a[rr][j], v)
            newk.append(v)
        newpk = perm[k]
        for rr in range(k + 1, 6):
            newpk = jnp.where(bi == iconst[rr], perm[rr], newpk)
        for rr in range(k + 1, 6):
            hit = bi == iconst[rr]
            for j in range(6):
                a[rr][j] = jnp.where(hit, rowk_old[j], a[rr][j])
            perm[rr] = jnp.where(hit, permk_old, perm[rr])
        a[k] = newk
        perm[k] = newpk
        xp = a[k][k]
        nz = xp != jnp.zeros((1, 1), jnp.float32)
        for rr in range(k + 1, 6):
            a[rr][k] = jnp.where(nz, a[rr][k] / xp, a[rr][k])
        for rr in range(k + 1, 6):
            for j in range(k + 1, 6):
                a[rr][j] = a[rr][j] - a[rr][k] * a[k][j]

    zero = jnp.zeros((1, 1), jnp.float32)
    Bm = [[jnp.where(perm[j] == iconst[c], one, zero) for c in range(6)]
          for j in range(6)]
    y = [[None] * 6 for _ in range(6)]
    for c in range(6):
        for rr in range(6):
            v = Bm[rr][c]
            for j in range(rr):
                v = v - a[rr][j] * y[j][c]
            y[rr][c] = v
    xs = [[None] * 6 for _ in range(6)]
    for c in range(6):
        for rr in range(5, -1, -1):
            v = y[rr][c]
            for j in range(rr + 1, 6):
                v = v - a[rr][j] * xs[j][c]
            xs[rr][c] = v / a[rr][rr]

    trace = xs[0][0]
    for c in range(1, 6):
        trace = trace + xs[c][c]
    o_ref[...] = lax.pow(trace, jnp.full((1, 1), 0.5, jnp.float32))


_tc_solve = pl.pallas_call(
    _tc_solve_body,
    out_shape=jax.ShapeDtypeStruct((1, 1), jnp.float32),
    in_specs=[pl.BlockSpec((NW, NBINS), lambda: (0, 0))],
    out_specs=pl.BlockSpec((1, 1), lambda: (0, 0)),
)


def kernel(input, target):
    codes = _codes(jnp.swapaxes(input, 0, 1), jnp.swapaxes(target, 0, 1))
    hist = _sc_call(codes)
    out = _tc_solve(hist)
    return lax.stop_gradient(out[0, 0])
